# baseline (speedup 1.0000x reference)
import functools

import jax
import jax.numpy as jnp
from jax import lax
from jax.experimental import pallas as pl
from jax.experimental.pallas import tpu as pltpu

R = 16384
D = 512
H = 256
NS = 1024
NC = 64
NBG = 16
COMMIT = 0.25
BLK = 256

KS0, KS1 = 1832780943, 270669613
KC0, KC1 = 64467757, 2916123636
_M32 = 0xFFFFFFFF


def _gumbel(x1_init, k0, k1):
    ks = (k0, k1, k0 ^ k1 ^ 0x1BD11BDA)
    rots = ((13, 15, 26, 6), (17, 29, 16, 24))
    x0 = None
    x1 = x1_init
    for i in range(5):
        for r in rots[i % 2]:
            x0 = (x1 + jnp.uint32(k0)) if x0 is None else (x0 + x1)
            x1 = (x1 << jnp.uint32(r)) | (x1 >> jnp.uint32(32 - r))
            x1 = x0 ^ x1
        x0 = x0 + jnp.uint32(ks[(i + 1) % 3] & _M32)
        x1 = x1 + jnp.uint32((ks[(i + 2) % 3] + i + 1) & _M32)
    bits = x0 ^ x1
    fb = (bits >> jnp.uint32(9)) | jnp.uint32(0x3F800000)
    f = lax.bitcast_convert_type(fb, jnp.float32) - jnp.float32(1.0)
    return -jnp.log(-jnp.log(f + jnp.float32(1e-20)))


def _vq_body(x_ref, ws_ref, wc_ref, pos_s_ref, pos_c_ref,
             x2s_ref, x2c_ref, w2s_ref, w2c_ref,
             bg_ref, m_ref, hb_ref,
             idxs_ref, idxc_ref, q_ref, cnts_ref, cntc_ref, stats_ref):
    j = pl.program_id(0)
    nsteps = pl.num_programs(0)

    @pl.when(j == 0)
    def _():
        cnts_ref[...] = jnp.zeros_like(cnts_ref)
        cntc_ref[...] = jnp.zeros_like(cntc_ref)
        stats_ref[0] = jnp.float32(0.0)
        stats_ref[1] = jnp.float32(0.0)
        stats_ref[2] = jnp.float32(0.0)
        stats_ref[3] = jnp.float32(0.0)

    x = x_ref[...]
    xs = x[:, :H]
    xc = x[:, H:]
    ws = ws_ref[...]
    wc = wc_ref[...]
    m = m_ref[...]
    has_bg = hb_ref[0] > jnp.float32(0.5)

    dots = lax.dot_general(xs, ws, (((1,), (1,)), ((), ())),
                           preferred_element_type=jnp.float32)
    dist_s = ((x2s_ref[...][:, None] + w2s_ref[...][None, :])
              - jnp.float32(2.0) * dots)
    iota_s = lax.broadcasted_iota(jnp.int32, (BLK, NS), 1)

    def _mask_s(dd):
        bg = bg_ref[...]
        fg = iota_s >= NBG
        allowed = jnp.logical_xor(fg, bg[:, None] > jnp.float32(0.5))
        return jnp.where(allowed, dd, jnp.float32(jnp.inf))

    dist_s = lax.cond(has_bg, _mask_s, lambda dd: dd, dist_s)
    x1s = pos_s_ref[...] + (jnp.uint32(KS1)
                            + (j * (BLK * NS)).astype(jnp.uint32))
    score_s = _gumbel(x1s, KS0, KS1) - dist_s
    mx = jnp.max(score_s, axis=1)
    idx_s = jnp.min(jnp.where(score_s == mx[:, None], iota_s, jnp.int32(NS)),
                    axis=1)
    oh_s = jnp.where(iota_s == idx_s[:, None], jnp.float32(1.0),
                     jnp.float32(0.0))
    cnts_ref[...] += lax.dot_general(m[None, :], oh_s,
                                     (((1,), (0,)), ((), ())),
                                     preferred_element_type=jnp.float32)[0]
    q_s = lax.dot_general(oh_s, ws, (((1,), (0,)), ((), ())),
                          preferred_element_type=jnp.float32)

    dotc = lax.dot_general(xc, wc, (((1,), (1,)), ((), ())),
                           preferred_element_type=jnp.float32)
    dist_c = ((x2c_ref[...][:, None] + w2c_ref[...][None, :])
              - jnp.float32(2.0) * dotc)
    iota_c = lax.broadcasted_iota(jnp.int32, (BLK, NC), 1)
    x1c = pos_c_ref[...] + (jnp.uint32(KC1)
                            + (j * (BLK * NC)).astype(jnp.uint32))
    score_c = _gumbel(x1c, KC0, KC1) - dist_c
    mxc = jnp.max(score_c, axis=1)
    idx_c = jnp.min(jnp.where(score_c == mxc[:, None], iota_c, jnp.int32(NC)),
                    axis=1)
    iota_c128 = lax.broadcasted_iota(jnp.int32, (BLK, 128), 1)
    oh_c128 = jnp.where(iota_c128 == idx_c[:, None], jnp.float32(1.0),
                        jnp.float32(0.0))
    cntc_ref[...] += lax.dot_general(m[None, :], oh_c128,
                                     (((1,), (0,)), ((), ())),
                                     preferred_element_type=jnp.float32)[0]
    q_c = lax.dot_general(oh_c128[:, :NC], wc, (((1,), (0,)), ((), ())),
                          preferred_element_type=jnp.float32)

    idxs_ref[...] = idx_s
    idxc_ref[...] = idx_c
    q_ref[:, :H] = q_s
    q_ref[:, H:] = q_c
    ds = q_s - xs
    dc = q_c - xc
    stats_ref[0] += jnp.sum(ds * ds) + jnp.sum(dc * dc)
    stats_ref[3] += jnp.sum(m)

    @pl.when(j == nsteps - 1)
    def _():
        den = jnp.maximum(stats_ref[3], jnp.float32(1.0))
        ps = cnts_ref[...] / den
        ent_s = jnp.sum(ps * jnp.log(ps + jnp.float32(1e-10)))
        stats_ref[1] = jnp.exp(-ent_s)
        pc = cntc_ref[...] / den
        ent_c = jnp.sum(pc * jnp.log(pc + jnp.float32(1e-10)))
        stats_ref[2] = jnp.exp(-ent_c)
        stats_ref[0] = stats_ref[0] * jnp.float32((1.0 + COMMIT) / (R * D))


@functools.partial(jax.jit, static_argnames=("interpret",))
def _run(flat, mvec, w_shape, w_color, interpret=False):
    x2s = jnp.sum(flat[:, :H] ** 2, axis=1)
    x2c = jnp.sum(flat[:, H:] ** 2, axis=1)
    w2s = jnp.sum(w_shape ** 2, axis=1)
    w2c = jnp.sum(w_color ** 2, axis=1)
    bg = (jnp.sum(jnp.abs(flat), axis=-1) < 1e-6).astype(jnp.float32)
    hb = jnp.max(bg, keepdims=True)

    pos_s = (jnp.arange(BLK, dtype=jnp.uint32)[:, None] * NS
             + jnp.arange(NS, dtype=jnp.uint32)[None, :])
    pos_c = (jnp.arange(BLK, dtype=jnp.uint32)[:, None] * NC
             + jnp.arange(NC, dtype=jnp.uint32)[None, :])

    idx_s, idx_c, q, cnts, cntc, stats = pl.pallas_call(
        _vq_body,
        grid=(R // BLK,),
        in_specs=[
            pl.BlockSpec((BLK, D), lambda j: (j, 0)),
            pl.BlockSpec((NS, H), lambda j: (0, 0)),
            pl.BlockSpec((NC, H), lambda j: (0, 0)),
            pl.BlockSpec((BLK, NS), lambda j: (0, 0)),
            pl.BlockSpec((BLK, NC), lambda j: (0, 0)),
            pl.BlockSpec((BLK,), lambda j: (j,)),
            pl.BlockSpec((BLK,), lambda j: (j,)),
            pl.BlockSpec((NS,), lambda j: (0,)),
            pl.BlockSpec((NC,), lambda j: (0,)),
            pl.BlockSpec((BLK,), lambda j: (j,)),
            pl.BlockSpec((BLK,), lambda j: (j,)),
            pl.BlockSpec(memory_space=pltpu.SMEM),
        ],
        out_specs=[
            pl.BlockSpec((BLK,), lambda j: (j,)),
            pl.BlockSpec((BLK,), lambda j: (j,)),
            pl.BlockSpec((BLK, D), lambda j: (j, 0)),
            pl.BlockSpec((NS,), lambda j: (0,)),
            pl.BlockSpec((128,), lambda j: (0,)),
            pl.BlockSpec(memory_space=pltpu.SMEM),
        ],
        out_shape=[
            jax.ShapeDtypeStruct((R,), jnp.int32),
            jax.ShapeDtypeStruct((R,), jnp.int32),
            jax.ShapeDtypeStruct((R, D), jnp.float32),
            jax.ShapeDtypeStruct((NS,), jnp.float32),
            jax.ShapeDtypeStruct((128,), jnp.float32),
            jax.ShapeDtypeStruct((4,), jnp.float32),
        ],
        interpret=interpret,
    )(flat, w_shape, w_color, pos_s, pos_c, x2s, x2c, w2s, w2c, bg, mvec, hb)
    return idx_s, idx_c, q, stats


def _forward(inputs, valid_mask, w_shape, w_color, interpret):
    flat = inputs.reshape(R, D)
    mvec = valid_mask.reshape(R).astype(jnp.float32)
    idx_s, idx_c, q, stats = _run(flat, mvec, w_shape, w_color,
                                  interpret=interpret)
    q_st = q.reshape(inputs.shape)
    return q_st, stats[0], stats[1], stats[2], idx_s, idx_c


def kernel(inputs, valid_mask, w_shape, w_color):
    return _forward(inputs, valid_mask, w_shape, w_color, False)


# speedup vs baseline: 1.0841x; 1.0841x over previous
import functools

import jax
import jax.numpy as jnp
from jax import lax
from jax.experimental import pallas as pl
from jax.experimental.pallas import tpu as pltpu

R = 16384
D = 512
H = 256
NS = 1024
NC = 64
NBG = 16
COMMIT = 0.25
BLK = 1024

KS0, KS1 = 1832780943, 270669613
KC0, KC1 = 64467757, 2916123636
_M32 = 0xFFFFFFFF


def _gumbel(x1_init, k0, k1):
    ks = (k0, k1, k0 ^ k1 ^ 0x1BD11BDA)
    rots = ((13, 15, 26, 6), (17, 29, 16, 24))
    x0 = None
    x1 = x1_init
    for i in range(5):
        for r in rots[i % 2]:
            x0 = (x1 + jnp.uint32(k0)) if x0 is None else (x0 + x1)
            x1 = (x1 << jnp.uint32(r)) | (x1 >> jnp.uint32(32 - r))
            x1 = x0 ^ x1
        x0 = x0 + jnp.uint32(ks[(i + 1) % 3] & _M32)
        x1 = x1 + jnp.uint32((ks[(i + 2) % 3] + i + 1) & _M32)
    bits = x0 ^ x1
    fb = (bits >> jnp.uint32(9)) | jnp.uint32(0x3F800000)
    f = lax.bitcast_convert_type(fb, jnp.float32) - jnp.float32(1.0)
    return -jnp.log(-jnp.log(f + jnp.float32(1e-20)))


def _vq_body(x_ref, ws_ref, wc_ref, pos_s_ref, pos_c_ref,
             x2s_ref, x2c_ref, w2s_ref, w2c_ref,
             bg_ref, m_ref, hb_ref,
             idxs_ref, idxc_ref, q_ref, cnts_ref, cntc_ref, stats_ref):
    j = pl.program_id(0)
    nsteps = pl.num_programs(0)

    @pl.when(j == 0)
    def _():
        cnts_ref[...] = jnp.zeros_like(cnts_ref)
        cntc_ref[...] = jnp.zeros_like(cntc_ref)
        stats_ref[0] = jnp.float32(0.0)
        stats_ref[1] = jnp.float32(0.0)
        stats_ref[2] = jnp.float32(0.0)
        stats_ref[3] = jnp.float32(0.0)

    x = x_ref[...]
    xs = x[:, :H]
    xc = x[:, H:]
    ws = ws_ref[...]
    wc = wc_ref[...]
    m = m_ref[...]
    has_bg = hb_ref[0] > jnp.float32(0.5)

    dots = lax.dot_general(xs, ws, (((1,), (1,)), ((), ())),
                           preferred_element_type=jnp.float32)
    dist_s = ((x2s_ref[...][:, None] + w2s_ref[...][None, :])
              - jnp.float32(2.0) * dots)
    iota_s = lax.broadcasted_iota(jnp.int32, (BLK, NS), 1)

    def _mask_s(dd):
        bg = bg_ref[...]
        fg = iota_s >= NBG
        allowed = jnp.logical_xor(fg, bg[:, None] > jnp.float32(0.5))
        return jnp.where(allowed, dd, jnp.float32(jnp.inf))

    dist_s = lax.cond(has_bg, _mask_s, lambda dd: dd, dist_s)
    x1s = pos_s_ref[...] + (jnp.uint32(KS1)
                            + (j * (BLK * NS)).astype(jnp.uint32))
    score_s = _gumbel(x1s, KS0, KS1) - dist_s
    mx = jnp.max(score_s, axis=1)
    idx_s = jnp.min(jnp.where(score_s == mx[:, None], iota_s, jnp.int32(NS)),
                    axis=1)
    oh_s = jnp.where(iota_s == idx_s[:, None], jnp.float32(1.0),
                     jnp.float32(0.0))
    cnts_ref[...] += lax.dot_general(m[None, :], oh_s,
                                     (((1,), (0,)), ((), ())),
                                     preferred_element_type=jnp.float32)[0]
    q_s = lax.dot_general(oh_s, ws, (((1,), (0,)), ((), ())),
                          preferred_element_type=jnp.float32)

    dotc = lax.dot_general(xc, wc, (((1,), (1,)), ((), ())),
                           preferred_element_type=jnp.float32)
    dist_c = ((x2c_ref[...][:, None] + w2c_ref[...][None, :])
              - jnp.float32(2.0) * dotc)
    iota_c = lax.broadcasted_iota(jnp.int32, (BLK, NC), 1)
    x1c = pos_c_ref[...] + (jnp.uint32(KC1)
                            + (j * (BLK * NC)).astype(jnp.uint32))
    score_c = _gumbel(x1c, KC0, KC1) - dist_c
    mxc = jnp.max(score_c, axis=1)
    idx_c = jnp.min(jnp.where(score_c == mxc[:, None], iota_c, jnp.int32(NC)),
                    axis=1)
    iota_c128 = lax.broadcasted_iota(jnp.int32, (BLK, 128), 1)
    oh_c128 = jnp.where(iota_c128 == idx_c[:, None], jnp.float32(1.0),
                        jnp.float32(0.0))
    cntc_ref[...] += lax.dot_general(m[None, :], oh_c128,
                                     (((1,), (0,)), ((), ())),
                                     preferred_element_type=jnp.float32)[0]
    q_c = lax.dot_general(oh_c128[:, :NC], wc, (((1,), (0,)), ((), ())),
                          preferred_element_type=jnp.float32)

    idxs_ref[...] = idx_s
    idxc_ref[...] = idx_c
    q_ref[:, :H] = q_s
    q_ref[:, H:] = q_c
    ds = q_s - xs
    dc = q_c - xc
    stats_ref[0] += jnp.sum(ds * ds) + jnp.sum(dc * dc)
    stats_ref[3] += jnp.sum(m)

    @pl.when(j == nsteps - 1)
    def _():
        den = jnp.maximum(stats_ref[3], jnp.float32(1.0))
        ps = cnts_ref[...] / den
        ent_s = jnp.sum(ps * jnp.log(ps + jnp.float32(1e-10)))
        stats_ref[1] = jnp.exp(-ent_s)
        pc = cntc_ref[...] / den
        ent_c = jnp.sum(pc * jnp.log(pc + jnp.float32(1e-10)))
        stats_ref[2] = jnp.exp(-ent_c)
        stats_ref[0] = stats_ref[0] * jnp.float32((1.0 + COMMIT) / (R * D))


@functools.partial(jax.jit, static_argnames=("interpret",))
def _run(flat, mvec, w_shape, w_color, interpret=False):
    x2s = jnp.sum(flat[:, :H] ** 2, axis=1)
    x2c = jnp.sum(flat[:, H:] ** 2, axis=1)
    w2s = jnp.sum(w_shape ** 2, axis=1)
    w2c = jnp.sum(w_color ** 2, axis=1)
    bg = (jnp.sum(jnp.abs(flat), axis=-1) < 1e-6).astype(jnp.float32)
    hb = jnp.max(bg, keepdims=True)

    pos_s = (jnp.arange(BLK, dtype=jnp.uint32)[:, None] * NS
             + jnp.arange(NS, dtype=jnp.uint32)[None, :])
    pos_c = (jnp.arange(BLK, dtype=jnp.uint32)[:, None] * NC
             + jnp.arange(NC, dtype=jnp.uint32)[None, :])

    idx_s, idx_c, q, cnts, cntc, stats = pl.pallas_call(
        _vq_body,
        grid=(R // BLK,),
        in_specs=[
            pl.BlockSpec((BLK, D), lambda j: (j, 0)),
            pl.BlockSpec((NS, H), lambda j: (0, 0)),
            pl.BlockSpec((NC, H), lambda j: (0, 0)),
            pl.BlockSpec((BLK, NS), lambda j: (0, 0)),
            pl.BlockSpec((BLK, NC), lambda j: (0, 0)),
            pl.BlockSpec((BLK,), lambda j: (j,)),
            pl.BlockSpec((BLK,), lambda j: (j,)),
            pl.BlockSpec((NS,), lambda j: (0,)),
            pl.BlockSpec((NC,), lambda j: (0,)),
            pl.BlockSpec((BLK,), lambda j: (j,)),
            pl.BlockSpec((BLK,), lambda j: (j,)),
            pl.BlockSpec(memory_space=pltpu.SMEM),
        ],
        out_specs=[
            pl.BlockSpec((BLK,), lambda j: (j,)),
            pl.BlockSpec((BLK,), lambda j: (j,)),
            pl.BlockSpec((BLK, D), lambda j: (j, 0)),
            pl.BlockSpec((NS,), lambda j: (0,)),
            pl.BlockSpec((128,), lambda j: (0,)),
            pl.BlockSpec(memory_space=pltpu.SMEM),
        ],
        out_shape=[
            jax.ShapeDtypeStruct((R,), jnp.int32),
            jax.ShapeDtypeStruct((R,), jnp.int32),
            jax.ShapeDtypeStruct((R, D), jnp.float32),
            jax.ShapeDtypeStruct((NS,), jnp.float32),
            jax.ShapeDtypeStruct((128,), jnp.float32),
            jax.ShapeDtypeStruct((4,), jnp.float32),
        ],
        interpret=interpret,
    )(flat, w_shape, w_color, pos_s, pos_c, x2s, x2c, w2s, w2c, bg, mvec, hb)
    return idx_s, idx_c, q, stats


def _forward(inputs, valid_mask, w_shape, w_color, interpret):
    flat = inputs.reshape(R, D)
    mvec = valid_mask.reshape(R).astype(jnp.float32)
    idx_s, idx_c, q, stats = _run(flat, mvec, w_shape, w_color,
                                  interpret=interpret)
    q_st = q.reshape(inputs.shape)
    return q_st, stats[0], stats[1], stats[2], idx_s, idx_c


def kernel(inputs, valid_mask, w_shape, w_color):
    return _forward(inputs, valid_mask, w_shape, w_color, False)
